# baseline (speedup 1.0000x reference)
import functools

import jax
import jax.numpy as jnp
from jax import lax
from jax.experimental import pallas as pl
from jax.experimental.pallas import tpu as pltpu
from jax.experimental.pallas import tpu_sc as plsc

N = 10000
E = 320000
NC = 2
NS = 16
NW = NC * NS
CHUNK = 128
E_TOT = E + N
K = 84
NB = 3
BLK = 28
E_PAD = NW * K * CHUNK
NROWS = 10112
R = NROWS // NS

CA = 96
KA = 108
NBA = 3
BLKA = 36
E_PAD_A = NW * KA * CA


def _sc_mesh():
    return plsc.VectorSubcoreMesh(core_axis_name="c", subcore_axis_name="s")


def _deg_sc(dst3, ones, zcol):
    @functools.partial(
        pl.kernel,
        out_type=jax.ShapeDtypeStruct((NC * NROWS, 1), jnp.float32),
        mesh=_sc_mesh(),
        scratch_types=[
            pltpu.VMEM((K, CHUNK), jnp.int32),
            pltpu.VMEM((CHUNK, 1), jnp.float32),
            pltpu.VMEM_SHARED((NROWS, 1), jnp.float32),
            pltpu.SemaphoreType.DMA,
        ],
    )
    def k(dst_hbm, ones_hbm, z_hbm, out_hbm, dst_v, ones_v, acc, sem):
        c = lax.axis_index("c")
        s = lax.axis_index("s")
        w = c * NS + s
        pltpu.sync_copy(dst_hbm.at[w], dst_v)
        pltpu.sync_copy(ones_hbm, ones_v)

        @pl.when(s == 0)
        def _():
            pltpu.sync_copy(z_hbm, acc)

        plsc.subcore_barrier()

        def body(j, carry):
            pltpu.async_copy(ones_v, acc.at[dst_v.at[j]], sem, add=True)

            @pl.when(j > 2)
            def _():
                pltpu.make_async_copy(ones_v, acc.at[dst_v.at[j - 3]], sem).wait()

            return carry

        lax.fori_loop(0, K, body, 0)

        def drain(j, carry):
            pltpu.make_async_copy(ones_v, acc.at[dst_v.at[j]], sem).wait()
            return carry

        lax.fori_loop(K - 3, K, drain, 0)
        plsc.subcore_barrier()

        @pl.when(s == 0)
        def _():
            pltpu.sync_copy(acc, out_hbm.at[pl.ds(c * NROWS, NROWS)])

    return k(dst3, ones, zcol)


def _agg_sc(g, src4, dst4, zrows, ch):
    @functools.partial(
        pl.kernel,
        out_type=jax.ShapeDtypeStruct((NC * NROWS, ch), jnp.float32),
        mesh=_sc_mesh(),
        scratch_types=[
            pltpu.VMEM((BLKA, CA), jnp.int32),
            pltpu.VMEM((BLKA, CA), jnp.int32),
            pltpu.VMEM((CA, ch), jnp.float32),
            pltpu.VMEM((CA, ch), jnp.float32),
            pltpu.VMEM((CA, ch), jnp.float32),
            pltpu.VMEM_SHARED((NROWS, ch), jnp.float32),
            pltpu.SemaphoreType.DMA,
            pltpu.SemaphoreType.DMA,
            pltpu.SemaphoreType.DMA,
        ],
    )
    def k(g_hbm, src_hbm, dst_hbm, z_hbm, out_hbm,
          src_v, dst_v, rows0, rows1, rows2, acc, sem0, sem1, sem2):
        c = lax.axis_index("c")
        s = lax.axis_index("s")
        w = c * NS + s
        pltpu.sync_copy(z_hbm.at[pl.ds(s * R, R)], acc.at[pl.ds(s * R, R)])
        plsc.subcore_barrier()

        bufs = ((rows0, sem0), (rows1, sem1), (rows2, sem2))

        def block(b, carry):
            pltpu.sync_copy(src_hbm.at[w * NBA + b], src_v)
            pltpu.sync_copy(dst_hbm.at[w * NBA + b], dst_v)
            for t, (rv, sm) in enumerate(bufs):
                pltpu.async_copy(g_hbm.at[src_v.at[t]], rv, sm)

            def body(jj, carry2):
                j0 = 3 * jj
                for t, (rv, sm) in enumerate(bufs):
                    pltpu.make_async_copy(g_hbm.at[src_v.at[j0 + t]], rv, sm).wait()
                    pltpu.sync_copy(rv, acc.at[dst_v.at[j0 + t]], add=True)

                    @pl.when(jj < BLKA // 3 - 1)
                    def _():
                        pltpu.async_copy(g_hbm.at[src_v.at[j0 + t + 3]], rv, sm)

                return carry2

            lax.fori_loop(0, BLKA // 3, body, 0)
            return carry

        lax.fori_loop(0, NBA, block, 0)
        plsc.subcore_barrier()
        pltpu.sync_copy(acc.at[pl.ds(s * R, R)],
                        out_hbm.at[pl.ds(c * NROWS + s * R, R)])

    return k(g, src4, dst4, zrows)


TB = 632
TG = NROWS // TB


def _tc1(xp, w1, degs):
    def body(x_ref, w_ref, d0_ref, d1_ref, g_ref, dinv_ref):
        deg = d0_ref[...] + d1_ref[...]
        dinv = jnp.where(deg > 0, lax.rsqrt(deg), 0.0)
        dinv_ref[...] = dinv
        h = jnp.dot(x_ref[...], w_ref[...], preferred_element_type=jnp.float32)
        g_ref[...] = h * dinv

    return pl.pallas_call(
        body,
        grid=(TG,),
        in_specs=[
            pl.BlockSpec((TB, 128), lambda i: (i, 0)),
            pl.BlockSpec((128, 128), lambda i: (0, 0)),
            pl.BlockSpec((TB, 1), lambda i: (i, 0)),
            pl.BlockSpec((TB, 1), lambda i: (TG + i, 0)),
        ],
        out_specs=(
            pl.BlockSpec((TB, 128), lambda i: (i, 0)),
            pl.BlockSpec((TB, 1), lambda i: (i, 0)),
        ),
        out_shape=(
            jax.ShapeDtypeStruct((NROWS, 128), jnp.float32),
            jax.ShapeDtypeStruct((NROWS, 1), jnp.float32),
        ),
    )(xp, w1, degs, degs)


def _tc2(p, dinv, w2, b1):
    def body(p0_ref, p1_ref, dinv_ref, w_ref, b_ref, g_ref):
        din = dinv_ref[...]
        h1 = jnp.maximum((p0_ref[...] + p1_ref[...]) * din + b_ref[...], 0.0)
        g2 = jnp.dot(h1, w_ref[...], preferred_element_type=jnp.float32)
        g_ref[:, 0:64] = g2 * din
        g_ref[:, 64:128] = jnp.zeros((TB, 64), jnp.float32)

    return pl.pallas_call(
        body,
        grid=(TG,),
        in_specs=[
            pl.BlockSpec((TB, 128), lambda i: (i, 0)),
            pl.BlockSpec((TB, 128), lambda i: (TG + i, 0)),
            pl.BlockSpec((TB, 1), lambda i: (i, 0)),
            pl.BlockSpec((128, 64), lambda i: (0, 0)),
            pl.BlockSpec((1, 128), lambda i: (0, 0)),
        ],
        out_specs=pl.BlockSpec((TB, 128), lambda i: (i, 0)),
        out_shape=jax.ShapeDtypeStruct((NROWS, 128), jnp.float32),
    )(p, p, dinv, w2, b1)


def _tc3(q, dinv, b2, ch):
    def body(q0_ref, q1_ref, dinv_ref, b_ref, out_ref):
        agg = q0_ref[:, 0:ch] + q1_ref[:, 0:ch]
        out_ref[...] = jnp.maximum(agg * dinv_ref[...] + b_ref[...], 0.0)

    full = pl.pallas_call(
        body,
        grid=(TG,),
        in_specs=[
            pl.BlockSpec((TB, 128), lambda i: (i, 0)),
            pl.BlockSpec((TB, 128), lambda i: (TG + i, 0)),
            pl.BlockSpec((TB, 1), lambda i: (i, 0)),
            pl.BlockSpec((1, ch), lambda i: (0, 0)),
        ],
        out_specs=pl.BlockSpec((TB, ch), lambda i: (i, 0)),
        out_shape=jax.ShapeDtypeStruct((NROWS, ch), jnp.float32),
    )(q, q, dinv, b2)
    return full[0:N]


def kernel(x, edge_index, W1, b1, W2, b2):
    loop = jnp.arange(N, dtype=jnp.int32)
    src = jnp.concatenate([edge_index[0].astype(jnp.int32), loop])
    dst = jnp.concatenate([edge_index[1].astype(jnp.int32), loop])
    pad = N + jnp.arange(E_PAD - E_TOT, dtype=jnp.int32) % (NROWS - N)
    dst3 = jnp.concatenate([dst, pad]).reshape(NW, K, CHUNK)
    pad_a = N + jnp.arange(E_PAD_A - E_TOT, dtype=jnp.int32) % (NROWS - N)
    src4 = jnp.concatenate([src, pad_a]).reshape(NW * NBA, BLKA, CA)
    dst4 = jnp.concatenate([dst, pad_a]).reshape(NW * NBA, BLKA, CA)

    ones = jnp.ones((CHUNK, 1), jnp.float32)
    zcol = jnp.zeros((NROWS, 1), jnp.float32)
    z128 = jnp.zeros((NROWS, 128), jnp.float32)

    degs = _deg_sc(dst3, ones, zcol)
    xp = jnp.concatenate([x, jnp.zeros((NROWS - N, 128), jnp.float32)])
    g1, dinv = _tc1(xp, W1, degs)
    p = _agg_sc(g1, src4, dst4, z128, 128)
    g2 = _tc2(p, dinv, W2, b1.reshape(1, 128))
    q = _agg_sc(g2, src4, dst4, z128, 128)
    return _tc3(q, dinv, b2.reshape(1, 64), 64)


# speedup vs baseline: 1.0629x; 1.0629x over previous
import functools

import jax
import jax.numpy as jnp
from jax import lax
from jax.experimental import pallas as pl
from jax.experimental.pallas import tpu as pltpu
from jax.experimental.pallas import tpu_sc as plsc

N = 10000
E = 320000
NC = 2
NS = 16
NW = NC * NS
CHUNK = 128
E_TOT = E + N
K = 84
NB = 3
BLK = 28
E_PAD = NW * K * CHUNK
NROWS = 10112
R = NROWS // NS

CA = 96
KA = 108
NBA = 3
BLKA = 36
E_PAD_A = NW * KA * CA


def _sc_mesh():
    return plsc.VectorSubcoreMesh(core_axis_name="c", subcore_axis_name="s")


def _deg_sc(dst3, ones, zcol):
    @functools.partial(
        pl.kernel,
        out_type=jax.ShapeDtypeStruct((NC * NROWS, 1), jnp.float32),
        mesh=_sc_mesh(),
        scratch_types=[
            pltpu.VMEM((K, CHUNK), jnp.int32),
            pltpu.VMEM((CHUNK, 1), jnp.float32),
            pltpu.VMEM_SHARED((NROWS, 1), jnp.float32),
            pltpu.SemaphoreType.DMA,
        ],
    )
    def k(dst_hbm, ones_hbm, z_hbm, out_hbm, dst_v, ones_v, acc, sem):
        c = lax.axis_index("c")
        s = lax.axis_index("s")
        w = c * NS + s
        pltpu.sync_copy(dst_hbm.at[w], dst_v)
        pltpu.sync_copy(ones_hbm, ones_v)

        @pl.when(s == 0)
        def _():
            pltpu.sync_copy(z_hbm, acc)

        plsc.subcore_barrier()

        def body(j, carry):
            pltpu.async_copy(ones_v, acc.at[dst_v.at[j]], sem, add=True)

            @pl.when(j > 2)
            def _():
                pltpu.make_async_copy(ones_v, acc.at[dst_v.at[j - 3]], sem).wait()

            return carry

        lax.fori_loop(0, K, body, 0)

        def drain(j, carry):
            pltpu.make_async_copy(ones_v, acc.at[dst_v.at[j]], sem).wait()
            return carry

        lax.fori_loop(K - 3, K, drain, 0)
        plsc.subcore_barrier()

        @pl.when(s == 0)
        def _():
            pltpu.sync_copy(acc, out_hbm.at[pl.ds(c * NROWS, NROWS)])

    return k(dst3, ones, zcol)


def _agg_sc(g, src4, dst4, zrows, ch):
    @functools.partial(
        pl.kernel,
        out_type=jax.ShapeDtypeStruct((NC * NROWS, ch), jnp.float32),
        mesh=_sc_mesh(),
        scratch_types=[
            pltpu.VMEM((BLKA, CA), jnp.int32),
            pltpu.VMEM((BLKA, CA), jnp.int32),
            pltpu.VMEM((CA, ch), jnp.float32),
            pltpu.VMEM((CA, ch), jnp.float32),
            pltpu.VMEM((CA, ch), jnp.float32),
            pltpu.VMEM_SHARED((NROWS, ch), jnp.float32),
            pltpu.SemaphoreType.DMA,
            pltpu.SemaphoreType.DMA,
            pltpu.SemaphoreType.DMA,
        ],
    )
    def k(g_hbm, src_hbm, dst_hbm, z_hbm, out_hbm,
          src_v, dst_v, rows0, rows1, rows2, acc, sem0, sem1, sem2):
        c = lax.axis_index("c")
        s = lax.axis_index("s")
        w = c * NS + s
        pltpu.sync_copy(z_hbm.at[pl.ds(s * R, R)], acc.at[pl.ds(s * R, R)])
        plsc.subcore_barrier()

        bufs = ((rows0, sem0), (rows1, sem1), (rows2, sem2))

        def block(b, carry):
            pltpu.sync_copy(src_hbm.at[w * NBA + b], src_v)
            pltpu.sync_copy(dst_hbm.at[w * NBA + b], dst_v)
            for t, (rv, sm) in enumerate(bufs):
                pltpu.async_copy(g_hbm.at[src_v.at[t]], rv, sm)

            def body(jj, carry2):
                j0 = 3 * jj
                for t, (rv, sm) in enumerate(bufs):
                    pltpu.make_async_copy(g_hbm.at[src_v.at[j0 + t]], rv, sm).wait()
                    pltpu.sync_copy(rv, acc.at[dst_v.at[j0 + t]], add=True)

                    @pl.when(jj < BLKA // 3 - 1)
                    def _():
                        pltpu.async_copy(g_hbm.at[src_v.at[j0 + t + 3]], rv, sm)

                return carry2

            lax.fori_loop(0, BLKA // 3, body, 0)
            return carry

        lax.fori_loop(0, NBA, block, 0)
        plsc.subcore_barrier()
        pltpu.sync_copy(acc.at[pl.ds(s * R, R)],
                        out_hbm.at[pl.ds(c * NROWS + s * R, R)])

    return k(g, src4, dst4, zrows)


def _tc1(x, w1, degs):
    def body(x_ref, w_ref, deg_ref, g_ref, dinv_ref):
        deg = deg_ref[0:NROWS] + deg_ref[NROWS:2 * NROWS]
        dinv = lax.rsqrt(deg)
        dinv_ref[...] = dinv
        h = jnp.dot(x_ref[...], w_ref[...], preferred_element_type=jnp.float32)
        g_ref[0:N, :] = h * dinv[0:N]
        g_ref[N:NROWS, :] = jnp.zeros((NROWS - N, w_ref.shape[1]), jnp.float32)

    return pl.pallas_call(
        body,
        out_shape=(
            jax.ShapeDtypeStruct((NROWS, 128), jnp.float32),
            jax.ShapeDtypeStruct((NROWS, 1), jnp.float32),
        ),
    )(x, w1, degs)


def _tc2(p, dinv, w2, b1):
    def body(p_ref, dinv_ref, w_ref, b_ref, g_ref):
        agg = p_ref[0:N] + p_ref[NROWS:NROWS + N]
        din = dinv_ref[0:N]
        h1 = jnp.maximum(agg * din + b_ref[...], 0.0)
        g2 = jnp.dot(h1, w_ref[...], preferred_element_type=jnp.float32)
        g_ref[0:N, 0:64] = g2 * din
        g_ref[0:N, 64:128] = jnp.zeros((N, 64), jnp.float32)
        g_ref[N:NROWS, :] = jnp.zeros((NROWS - N, 128), jnp.float32)

    return pl.pallas_call(
        body,
        out_shape=jax.ShapeDtypeStruct((NROWS, 128), jnp.float32),
    )(p, dinv, w2, b1)


def _tc3(q, dinv, b2, ch):
    def body(q_ref, dinv_ref, b_ref, out_ref):
        agg = q_ref[0:N, 0:ch] + q_ref[NROWS:NROWS + N, 0:ch]
        out_ref[...] = jnp.maximum(agg * dinv_ref[0:N] + b_ref[...], 0.0)

    return pl.pallas_call(
        body,
        out_shape=jax.ShapeDtypeStruct((N, ch), jnp.float32),
    )(q, dinv, b2)


def kernel(x, edge_index, W1, b1, W2, b2):
    loop = jnp.arange(N, dtype=jnp.int32)
    src = jnp.concatenate([edge_index[0].astype(jnp.int32), loop])
    dst = jnp.concatenate([edge_index[1].astype(jnp.int32), loop])
    pad = N + jnp.arange(E_PAD - E_TOT, dtype=jnp.int32) % (NROWS - N)
    dst3 = jnp.concatenate([dst, pad]).reshape(NW, K, CHUNK)
    pad_a = N + jnp.arange(E_PAD_A - E_TOT, dtype=jnp.int32) % (NROWS - N)
    src4 = jnp.concatenate([src, pad_a]).reshape(NW * NBA, BLKA, CA)
    dst4 = jnp.concatenate([dst, pad_a]).reshape(NW * NBA, BLKA, CA)

    ones = jnp.ones((CHUNK, 1), jnp.float32)
    zcol = jnp.zeros((NROWS, 1), jnp.float32)
    z128 = jnp.zeros((NROWS, 128), jnp.float32)

    degs = _deg_sc(dst3, ones, zcol)
    g1, dinv = _tc1(x, W1, degs)
    p = _agg_sc(g1, src4, dst4, z128, 128)
    g2 = _tc2(p, dinv, W2, b1.reshape(1, 128))
    q = _agg_sc(g2, src4, dst4, z128, 128)
    return _tc3(q, dinv, b2.reshape(1, 64), 64)
